# Initial kernel scaffold; baseline (speedup 1.0000x reference)
import jax
import jax.numpy as jnp
from jax.experimental import pallas as pl


def kernel(x, edge_index, Ws, bs, gammas, betas):
    raise NotImplementedError("write your pallas kernel here")


# speedup vs baseline: 14.8076x; 14.8076x over previous
import functools
import math

import jax
import jax.numpy as jnp
import numpy as np
from jax import lax
from jax.experimental import pallas as pl
from jax.experimental.pallas import tpu as pltpu
from jax.experimental.pallas import tpu_sc as plsc

F32 = jnp.float32

NC = 2
NS = 16
LANES = 16
CH = 128


def _pad16(d: int) -> int:
    return ((d + 15) // 16) * 16


def _mesh():
    return plsc.VectorSubcoreMesh(core_axis_name="c", subcore_axis_name="s")


def _make_sc_degree(n, t):
    npad = n + 8
    rpt = n // NS

    @functools.partial(
        pl.kernel,
        out_type=jax.ShapeDtypeStruct((NC, n, LANES), F32),
        mesh=_mesh(),
        compiler_params=pltpu.CompilerParams(use_tc_tiling_on_sc=False),
        scratch_types=[
            pltpu.VMEM((t, CH), jnp.int32),
            pltpu.VMEM((CH, LANES), F32),
            pltpu.VMEM((rpt, LANES), F32),
            pltpu.VMEM_SHARED((npad, LANES), F32),
        ],
    )
    def deg_kernel(dst_hbm, zeros_hbm, ones_hbm, out_hbm,
                   dst_v, ones_v, bounce_v, acc_sh):
        c = lax.axis_index("c")
        s = lax.axis_index("s")
        wid = c * NS + s
        pltpu.sync_copy(dst_hbm.at[wid], dst_v)
        pltpu.sync_copy(ones_hbm, ones_v)
        sl = pl.ds(s * rpt, rpt)
        pltpu.sync_copy(zeros_hbm.at[sl], bounce_v)
        pltpu.sync_copy(bounce_v, acc_sh.at[sl])
        plsc.subcore_barrier()

        def body(tt, carry):
            pltpu.sync_copy(ones_v, acc_sh.at[dst_v.at[tt]], add=True)
            return carry

        lax.fori_loop(0, t, body, 0)
        plsc.subcore_barrier()
        pltpu.sync_copy(acc_sh.at[sl], bounce_v)
        pltpu.sync_copy(bounce_v, out_hbm.at[c, sl])

    return deg_kernel


def _make_sc_agg(n, t, fp):
    npad = n + 8
    rpt = n // NS

    @functools.partial(
        pl.kernel,
        out_type=jax.ShapeDtypeStruct((NC, n, fp), F32),
        mesh=_mesh(),
        compiler_params=pltpu.CompilerParams(use_tc_tiling_on_sc=False),
        scratch_types=[
            pltpu.VMEM((t, CH), jnp.int32),
            pltpu.VMEM((t, CH), jnp.int32),
            pltpu.VMEM((CH, fp), F32),
            pltpu.VMEM((CH, fp), F32),
            pltpu.VMEM((rpt, fp), F32),
            pltpu.VMEM_SHARED((npad, fp), F32),
            pltpu.SemaphoreType.DMA,
            pltpu.SemaphoreType.DMA,
        ],
    )
    def agg_kernel(src_hbm, dst_hbm, hs_hbm, zeros_hbm, out_hbm,
                   src_v, dst_v, rows0, rows1, bounce_v, acc_sh, sem0, sem1):
        c = lax.axis_index("c")
        s = lax.axis_index("s")
        wid = c * NS + s
        pltpu.sync_copy(src_hbm.at[wid], src_v)
        pltpu.sync_copy(dst_hbm.at[wid], dst_v)
        sl = pl.ds(s * rpt, rpt)

        @pl.when(c == 0)
        def _():
            pltpu.sync_copy(hs_hbm.at[sl], bounce_v)

        @pl.when(c != 0)
        def _():
            pltpu.sync_copy(zeros_hbm.at[sl], bounce_v)

        pltpu.sync_copy(bounce_v, acc_sh.at[sl])
        plsc.subcore_barrier()

        def gather(tt, rows, sem):
            return pltpu.make_async_copy(hs_hbm.at[src_v.at[tt]], rows, sem)

        def scat(tt, rows):
            pltpu.sync_copy(rows, acc_sh.at[dst_v.at[tt]], add=True)

        gather(0, rows0, sem0).start()

        def body(i, carry):
            t0 = 2 * i
            gather(t0 + 1, rows1, sem1).start()
            gather(t0, rows0, sem0).wait()
            scat(t0, rows0)

            @pl.when(t0 + 2 < t)
            def _():
                gather(t0 + 2, rows0, sem0).start()

            gather(t0 + 1, rows1, sem1).wait()
            scat(t0 + 1, rows1)
            return carry

        lax.fori_loop(0, t // 2, body, 0)
        plsc.subcore_barrier()
        pltpu.sync_copy(acc_sh.at[sl], bounce_v)
        pltpu.sync_copy(bounce_v, out_hbm.at[c, sl])

    return agg_kernel


_RB = 2000


def _tc_first(d0, d1, x, w0):
    n, fin = x.shape
    fout = w0.shape[1]

    def body(d0_r, d1_r, x_r, w_r, dinv_r, hs_r):
        dinv = 1.0 / jnp.sqrt(1.0 + d0_r[...] + d1_r[...])
        h = jnp.dot(x_r[...], w_r[...], preferred_element_type=F32)
        dinv_r[...] = dinv
        hs_r[...] = h * dinv

    return pl.pallas_call(
        body,
        grid=(n // _RB,),
        in_specs=[
            pl.BlockSpec((_RB, 1), lambda i: (i, 0)),
            pl.BlockSpec((_RB, 1), lambda i: (i, 0)),
            pl.BlockSpec((_RB, fin), lambda i: (i, 0)),
            pl.BlockSpec((fin, fout), lambda i: (0, 0)),
        ],
        out_specs=[
            pl.BlockSpec((_RB, 1), lambda i: (i, 0)),
            pl.BlockSpec((_RB, fout), lambda i: (i, 0)),
        ],
        out_shape=[
            jax.ShapeDtypeStruct((n, 1), F32),
            jax.ShapeDtypeStruct((n, fout), F32),
        ],
    )(d0, d1, x, w0)


def _tc_mid(a0, a1, dinv, bvec, scale, beta, w):
    n, fin = a0.shape
    fout = w.shape[1]

    def body(a0_r, a1_r, dinv_r, b_r, s_r, bt_r, w_r, hs_r):
        dinv = dinv_r[...]
        y = (a0_r[...] + a1_r[...]) * dinv + b_r[...]
        act = jnp.where(y > 0, y, jnp.exp(y) - 1.0)
        z = act * s_r[...] + bt_r[...]
        h = jnp.dot(z, w_r[...], preferred_element_type=F32)
        hs_r[...] = h * dinv

    return pl.pallas_call(
        body,
        grid=(n // _RB,),
        in_specs=[
            pl.BlockSpec((_RB, fin), lambda i: (i, 0)),
            pl.BlockSpec((_RB, fin), lambda i: (i, 0)),
            pl.BlockSpec((_RB, 1), lambda i: (i, 0)),
            pl.BlockSpec((1, fin), lambda i: (0, 0)),
            pl.BlockSpec((1, fin), lambda i: (0, 0)),
            pl.BlockSpec((1, fin), lambda i: (0, 0)),
            pl.BlockSpec((fin, fout), lambda i: (0, 0)),
        ],
        out_specs=pl.BlockSpec((_RB, fout), lambda i: (i, 0)),
        out_shape=jax.ShapeDtypeStruct((n, fout), F32),
    )(a0, a1, dinv, bvec, scale, beta, w)


def _tc_final(a0, a1, dinv, bvec):
    n, fin = a0.shape

    def body(a0_r, a1_r, dinv_r, b_r, y_r):
        y_r[...] = (a0_r[...] + a1_r[...]) * dinv_r[...] + b_r[...]

    return pl.pallas_call(
        body,
        grid=(n // _RB,),
        in_specs=[
            pl.BlockSpec((_RB, fin), lambda i: (i, 0)),
            pl.BlockSpec((_RB, fin), lambda i: (i, 0)),
            pl.BlockSpec((_RB, 1), lambda i: (i, 0)),
            pl.BlockSpec((1, fin), lambda i: (0, 0)),
        ],
        out_specs=pl.BlockSpec((_RB, fin), lambda i: (i, 0)),
        out_shape=jax.ShapeDtypeStruct((n, fin), F32),
    )(a0, a1, dinv, bvec)


def kernel(x, edge_index, Ws, bs, gammas, betas):
    n = x.shape[0]
    e = edge_index.shape[1]
    nl = len(Ws)
    dims = [x.shape[1]] + [w.shape[1] for w in Ws]
    fps = [_pad16(d) for d in dims]

    t = math.ceil(e / (NC * NS * CH))
    if t % 2:
        t += 1
    e_pad = NC * NS * t * CH

    src = jnp.concatenate(
        [edge_index[0], jnp.zeros((e_pad - e,), edge_index.dtype)]
    ).reshape(NC * NS, t, CH)
    dst = jnp.concatenate(
        [edge_index[1], jnp.full((e_pad - e,), n, edge_index.dtype)]
    ).reshape(NC * NS, t, CH)

    degp = _make_sc_degree(n, t)(
        dst, jnp.zeros((n, LANES), F32), jnp.ones((CH, LANES), F32))
    d0 = degp[0, :, 0:1]
    d1 = degp[1, :, 0:1]

    xp = jnp.pad(x, ((0, 0), (0, fps[0] - dims[0])))
    Wp = [jnp.pad(Ws[i], ((0, fps[i] - dims[i]), (0, fps[i + 1] - dims[i + 1])))
          for i in range(nl)]
    bp = [jnp.pad(bs[i], (0, fps[i + 1] - dims[i + 1])).reshape(1, -1)
          for i in range(nl)]
    inv_bn = 1.0 / np.sqrt(1.0 + 1e-5)
    scalep = [(jnp.pad(gammas[i], (0, fps[i + 1] - dims[i + 1])) * inv_bn
               ).reshape(1, -1) for i in range(nl - 1)]
    betap = [jnp.pad(betas[i], (0, fps[i + 1] - dims[i + 1])).reshape(1, -1)
             for i in range(nl - 1)]

    dinv, hs = _tc_first(d0, d1, xp, Wp[0])

    for i in range(nl - 1):
        fp = fps[i + 1]
        agg = _make_sc_agg(n, t, fp)(src, dst, hs, jnp.zeros((n, fp), F32))
        hs = _tc_mid(agg[0], agg[1], dinv, bp[i], scalep[i], betap[i], Wp[i + 1])

    fp = fps[nl]
    agg = _make_sc_agg(n, t, fp)(src, dst, hs, jnp.zeros((n, fp), F32))
    y = _tc_final(agg[0], agg[1], dinv, bp[nl - 1])
    return y[:, :dims[nl]]


# speedup vs baseline: 15.8116x; 1.0678x over previous
import functools
import math

import jax
import jax.numpy as jnp
import numpy as np
from jax import lax
from jax.experimental import pallas as pl
from jax.experimental.pallas import tpu as pltpu
from jax.experimental.pallas import tpu_sc as plsc

F32 = jnp.float32

NC = 2
NS = 16
LANES = 16
CH = 128


def _pad16(d: int) -> int:
    return ((d + 15) // 16) * 16


def _mesh():
    return plsc.VectorSubcoreMesh(core_axis_name="c", subcore_axis_name="s")


def _make_sc_degree(n, t):
    npad = n + 8
    rpt = n // NS

    @functools.partial(
        pl.kernel,
        out_type=jax.ShapeDtypeStruct((NC, n, LANES), F32),
        mesh=_mesh(),
        compiler_params=pltpu.CompilerParams(use_tc_tiling_on_sc=False),
        scratch_types=[
            pltpu.VMEM((t, CH), jnp.int32),
            pltpu.VMEM((CH, LANES), F32),
            pltpu.VMEM((rpt, LANES), F32),
            pltpu.VMEM_SHARED((npad, LANES), F32),
        ],
    )
    def deg_kernel(dst_hbm, zeros_hbm, ones_hbm, out_hbm,
                   dst_v, ones_v, bounce_v, acc_sh):
        c = lax.axis_index("c")
        s = lax.axis_index("s")
        wid = c * NS + s
        pltpu.sync_copy(dst_hbm.at[wid], dst_v)
        pltpu.sync_copy(ones_hbm, ones_v)
        sl = pl.ds(s * rpt, rpt)
        pltpu.sync_copy(zeros_hbm.at[sl], bounce_v)
        pltpu.sync_copy(bounce_v, acc_sh.at[sl])
        plsc.subcore_barrier()

        def body(tt, carry):
            pltpu.sync_copy(ones_v, acc_sh.at[dst_v.at[tt]], add=True)
            return carry

        lax.fori_loop(0, t, body, 0)
        plsc.subcore_barrier()
        pltpu.sync_copy(acc_sh.at[sl], bounce_v)
        pltpu.sync_copy(bounce_v, out_hbm.at[c, sl])

    return deg_kernel


def _make_sc_agg(n, t, fp):
    npad = n + 8
    rpt = n // NS
    rb = rpt // 5
    nbp = rpt // rb

    @functools.partial(
        pl.kernel,
        out_type=jax.ShapeDtypeStruct((NC, n, fp), F32),
        mesh=_mesh(),
        compiler_params=pltpu.CompilerParams(use_tc_tiling_on_sc=False),
        scratch_types=[
            pltpu.VMEM((t, CH), jnp.int32),
            pltpu.VMEM((t, CH), jnp.int32),
            pltpu.VMEM((CH, fp), F32),
            pltpu.VMEM((CH, fp), F32),
            pltpu.VMEM((CH, fp), F32),
            pltpu.VMEM((CH, fp), F32),
            pltpu.VMEM((rb, fp), F32),
            pltpu.VMEM_SHARED((npad, fp), F32),
            pltpu.SemaphoreType.DMA,
            pltpu.SemaphoreType.DMA,
            pltpu.SemaphoreType.DMA,
            pltpu.SemaphoreType.DMA,
            pltpu.SemaphoreType.DMA,
            pltpu.SemaphoreType.DMA,
            pltpu.SemaphoreType.DMA,
            pltpu.SemaphoreType.DMA,
        ],
    )
    def agg_kernel(src_hbm, dst_hbm, hs_hbm, zeros_hbm, out_hbm,
                   src_v, dst_v, rows0, rows1, rows2, rows3, bounce_v, acc_sh,
                   g0, g1, g2, g3, s0, s1, s2, s3):
        c = lax.axis_index("c")
        s = lax.axis_index("s")
        wid = c * NS + s
        pltpu.sync_copy(src_hbm.at[wid], src_v)
        pltpu.sync_copy(dst_hbm.at[wid], dst_v)

        for k in range(nbp):
            slk = pl.ds(s * rpt + k * rb, rb)

            @pl.when(c == 0)
            def _():
                pltpu.sync_copy(hs_hbm.at[slk], bounce_v)

            @pl.when(c != 0)
            def _():
                pltpu.sync_copy(zeros_hbm.at[slk], bounce_v)

            pltpu.sync_copy(bounce_v, acc_sh.at[slk])
        plsc.subcore_barrier()

        rows = (rows0, rows1, rows2, rows3)
        gsem = (g0, g1, g2, g3)
        ssem = (s0, s1, s2, s3)

        def gather(tt, j):
            return pltpu.make_async_copy(hs_hbm.at[src_v.at[tt]], rows[j], gsem[j])

        def scat(tt, j):
            return pltpu.make_async_copy(rows[j], acc_sh.at[dst_v.at[tt]], ssem[j])

        for j in range(4):
            gather(j, j).start()
        for j in range(4):
            gather(j, j).wait()
            scat(j, j).start(add=True)

        def body(i, carry):
            base = 4 * i
            for j in range(4):
                scat(base - 4 + j, j).wait()
                gather(base + j, j).start()
            for j in range(4):
                gather(base + j, j).wait()
                scat(base + j, j).start(add=True)
            return carry

        lax.fori_loop(1, t // 4, body, 0)
        for j in range(4):
            scat(t - 4 + j, j).wait()
        plsc.subcore_barrier()
        for k in range(nbp):
            slk = pl.ds(s * rpt + k * rb, rb)
            pltpu.sync_copy(acc_sh.at[slk], bounce_v)
            pltpu.sync_copy(bounce_v, out_hbm.at[c, slk])

    return agg_kernel


_RB = 2000


def _tc_first(d0, d1, x, w0):
    n, fin = x.shape
    fout = w0.shape[1]

    def body(d0_r, d1_r, x_r, w_r, dinv_r, hs_r):
        dinv = 1.0 / jnp.sqrt(1.0 + d0_r[...] + d1_r[...])
        h = jnp.dot(x_r[...], w_r[...], preferred_element_type=F32)
        dinv_r[...] = dinv
        hs_r[...] = h * dinv

    return pl.pallas_call(
        body,
        grid=(n // _RB,),
        in_specs=[
            pl.BlockSpec((_RB, 1), lambda i: (i, 0)),
            pl.BlockSpec((_RB, 1), lambda i: (i, 0)),
            pl.BlockSpec((_RB, fin), lambda i: (i, 0)),
            pl.BlockSpec((fin, fout), lambda i: (0, 0)),
        ],
        out_specs=[
            pl.BlockSpec((_RB, 1), lambda i: (i, 0)),
            pl.BlockSpec((_RB, fout), lambda i: (i, 0)),
        ],
        out_shape=[
            jax.ShapeDtypeStruct((n, 1), F32),
            jax.ShapeDtypeStruct((n, fout), F32),
        ],
    )(d0, d1, x, w0)


def _tc_mid(agg, dinv, bvec, scale, beta, w):
    _, n, fin = agg.shape
    fout = w.shape[1]

    def body(agg_r, dinv_r, b_r, s_r, bt_r, w_r, hs_r):
        dinv = dinv_r[...]
        y = (agg_r[0] + agg_r[1]) * dinv + b_r[...]
        act = jnp.where(y > 0, y, jnp.exp(y) - 1.0)
        z = act * s_r[...] + bt_r[...]
        h = jnp.dot(z, w_r[...], preferred_element_type=F32)
        hs_r[...] = h * dinv

    return pl.pallas_call(
        body,
        grid=(n // _RB,),
        in_specs=[
            pl.BlockSpec((2, _RB, fin), lambda i: (0, i, 0)),
            pl.BlockSpec((_RB, 1), lambda i: (i, 0)),
            pl.BlockSpec((1, fin), lambda i: (0, 0)),
            pl.BlockSpec((1, fin), lambda i: (0, 0)),
            pl.BlockSpec((1, fin), lambda i: (0, 0)),
            pl.BlockSpec((fin, fout), lambda i: (0, 0)),
        ],
        out_specs=pl.BlockSpec((_RB, fout), lambda i: (i, 0)),
        out_shape=jax.ShapeDtypeStruct((n, fout), F32),
    )(agg, dinv, bvec, scale, beta, w)


def _tc_final(agg, dinv, bvec):
    _, n, fin = agg.shape

    def body(agg_r, dinv_r, b_r, y_r):
        y_r[...] = (agg_r[0] + agg_r[1]) * dinv_r[...] + b_r[...]

    return pl.pallas_call(
        body,
        grid=(n // _RB,),
        in_specs=[
            pl.BlockSpec((2, _RB, fin), lambda i: (0, i, 0)),
            pl.BlockSpec((_RB, 1), lambda i: (i, 0)),
            pl.BlockSpec((1, fin), lambda i: (0, 0)),
        ],
        out_specs=pl.BlockSpec((_RB, fin), lambda i: (i, 0)),
        out_shape=jax.ShapeDtypeStruct((n, fin), F32),
    )(agg, dinv, bvec)


def kernel(x, edge_index, Ws, bs, gammas, betas):
    n = x.shape[0]
    e = edge_index.shape[1]
    nl = len(Ws)
    dims = [x.shape[1]] + [w.shape[1] for w in Ws]
    fps = [_pad16(d) for d in dims]

    t = math.ceil(e / (NC * NS * CH))
    t = ((t + 3) // 4) * 4
    e_pad = NC * NS * t * CH

    src = jnp.concatenate(
        [edge_index[0], jnp.zeros((e_pad - e,), edge_index.dtype)]
    ).reshape(NC * NS, t, CH)
    dst = jnp.concatenate(
        [edge_index[1], jnp.full((e_pad - e,), n, edge_index.dtype)]
    ).reshape(NC * NS, t, CH)

    degp = _make_sc_degree(n, t)(
        dst, jnp.zeros((n, LANES), F32), jnp.ones((CH, LANES), F32))
    d0 = degp[0, :, 0:1]
    d1 = degp[1, :, 0:1]

    xp = jnp.pad(x, ((0, 0), (0, fps[0] - dims[0])))
    Wp = [jnp.pad(Ws[i], ((0, fps[i] - dims[i]), (0, fps[i + 1] - dims[i + 1])))
          for i in range(nl)]
    bp = [jnp.pad(bs[i], (0, fps[i + 1] - dims[i + 1])).reshape(1, -1)
          for i in range(nl)]
    inv_bn = 1.0 / np.sqrt(1.0 + 1e-5)
    scalep = [(jnp.pad(gammas[i], (0, fps[i + 1] - dims[i + 1])) * inv_bn
               ).reshape(1, -1) for i in range(nl - 1)]
    betap = [jnp.pad(betas[i], (0, fps[i + 1] - dims[i + 1])).reshape(1, -1)
             for i in range(nl - 1)]

    dinv, hs = _tc_first(d0, d1, xp, Wp[0])

    for i in range(nl - 1):
        fp = fps[i + 1]
        agg = _make_sc_agg(n, t, fp)(src, dst, hs, jnp.zeros((n, fp), F32))
        hs = _tc_mid(agg, dinv, bp[i], scalep[i], betap[i], Wp[i + 1])

    fp = fps[nl]
    agg = _make_sc_agg(n, t, fp)(src, dst, hs, jnp.zeros((n, fp), F32))
    y = _tc_final(agg, dinv, bp[nl - 1])
    return y[:, :dims[nl]]
